# initial kernel scaffold (unmeasured)
import jax
import jax.numpy as jnp
from jax import lax
from jax.experimental import pallas as pl
from jax.experimental.pallas import tpu as pltpu


def kernel(
    x,
):
    def body(*refs):
        pass

    out_shape = jax.ShapeDtypeStruct(..., jnp.float32)
    return pl.pallas_call(body, out_shape=out_shape)(...)


# baseline (device time: 51620 ns/iter reference)
import jax
import jax.numpy as jnp
from jax import lax
from jax.experimental import pallas as pl
from jax.experimental.pallas import tpu as pltpu

N_DEV = 4
N_CHUNKS = 8


def kernel(x):
    m, n = x.shape
    C = m // N_CHUNKS

    def body(x_ref, out_ref, lhalo, rhalo, stage, send_sems, recv_sems, copy_sems):
        my = lax.axis_index("i")
        left = (my - 1) % N_DEV
        right = (my + 1) % N_DEV

        barrier_sem = pltpu.get_barrier_semaphore()
        for nbr in [left, right]:
            pl.semaphore_signal(
                barrier_sem, inc=1,
                device_id=(nbr,), device_id_type=pl.DeviceIdType.MESH,
            )
        pl.semaphore_wait(barrier_sem, 2)

        send_r = pltpu.make_async_remote_copy(
            src_ref=x_ref.at[pl.ds(m - 1, 1), :],
            dst_ref=lhalo,
            send_sem=send_sems.at[0],
            recv_sem=recv_sems.at[0],
            device_id=(right,),
            device_id_type=pl.DeviceIdType.MESH,
        )
        send_r.start()
        send_l = pltpu.make_async_remote_copy(
            src_ref=x_ref.at[pl.ds(0, 1), :],
            dst_ref=rhalo,
            send_sem=send_sems.at[1],
            recv_sem=recv_sems.at[1],
            device_id=(left,),
            device_id_type=pl.DeviceIdType.MESH,
        )
        send_l.start()

        pending = []

        def emit(c, idx):
            slot = idx % 2
            if idx >= 2:
                pending[idx - 2].wait()
            lo = c * C
            if c == 0:
                send_r.wait_recv()
                row0 = (
                    0.25 * lhalo[:, :]
                    + 0.5 * x_ref[pl.ds(0, 1), :]
                    + 0.25 * x_ref[pl.ds(1, 1), :]
                )
                stage[slot, pl.ds(0, 1), :] = jnp.where(
                    my == 0, x_ref[pl.ds(0, 1), :], row0
                )
                stage[slot, pl.ds(1, C - 1), :] = (
                    0.25 * x_ref[pl.ds(0, C - 1), :]
                    + 0.5 * x_ref[pl.ds(1, C - 1), :]
                    + 0.25 * x_ref[pl.ds(2, C - 1), :]
                )
            elif c == N_CHUNKS - 1:
                send_l.wait_recv()
                stage[slot, pl.ds(0, C - 1), :] = (
                    0.25 * x_ref[pl.ds(lo - 1, C - 1), :]
                    + 0.5 * x_ref[pl.ds(lo, C - 1), :]
                    + 0.25 * x_ref[pl.ds(lo + 1, C - 1), :]
                )
                rowm = (
                    0.25 * x_ref[pl.ds(m - 2, 1), :]
                    + 0.5 * x_ref[pl.ds(m - 1, 1), :]
                    + 0.25 * rhalo[:, :]
                )
                stage[slot, pl.ds(C - 1, 1), :] = jnp.where(
                    my == N_DEV - 1, x_ref[pl.ds(m - 1, 1), :], rowm
                )
            else:
                stage[slot, :, :] = (
                    0.25 * x_ref[pl.ds(lo - 1, C), :]
                    + 0.5 * x_ref[pl.ds(lo, C), :]
                    + 0.25 * x_ref[pl.ds(lo + 1, C), :]
                )
            cp = pltpu.make_async_copy(
                stage.at[slot], out_ref.at[pl.ds(lo, C), :], copy_sems.at[slot]
            )
            cp.start()
            pending.append(cp)

        for idx, c in enumerate([1, 2, 3, 4, 5, 6, 0, N_CHUNKS - 1]):
            emit(c, idx)

        pending[-2].wait()
        pending[-1].wait()
        send_r.wait_send()
        send_l.wait_send()

    return pl.pallas_call(
        body,
        out_shape=jax.ShapeDtypeStruct((m, n), x.dtype),
        in_specs=[pl.BlockSpec(memory_space=pltpu.VMEM)],
        out_specs=pl.BlockSpec(memory_space=pltpu.MemorySpace.HBM),
        scratch_shapes=[
            pltpu.VMEM((1, n), x.dtype),
            pltpu.VMEM((1, n), x.dtype),
            pltpu.VMEM((2, C, n), x.dtype),
            pltpu.SemaphoreType.DMA((2,)),
            pltpu.SemaphoreType.DMA((2,)),
            pltpu.SemaphoreType.DMA((2,)),
        ],
        compiler_params=pltpu.CompilerParams(
            collective_id=0,
            vmem_limit_bytes=63 * 1024 * 1024,
        ),
    )(x)


# device time: 50646 ns/iter; 1.0192x vs baseline; 1.0192x over previous
import jax
import jax.numpy as jnp
from jax import lax
from jax.experimental import pallas as pl
from jax.experimental.pallas import tpu as pltpu

N_DEV = 4
N_CHUNKS = 8
A = 8


def kernel(x):
    m, n = x.shape
    C = m // N_CHUNKS
    order = list(range(1, N_CHUNKS - 1)) + [0, N_CHUNKS - 1]

    def body(x_ref, out_ref, lhalo, rhalo, in_buf, stage, edge_buf, send_row,
             send_sems, recv_sems, in_sems, out_sems, edge_sems):
        my = lax.axis_index("i")
        left = (my - 1) % N_DEV
        right = (my + 1) % N_DEV

        first_cp = pltpu.make_async_copy(
            x_ref.at[pl.ds(0, A), :], edge_buf.at[0], edge_sems.at[0]
        )
        last_cp = pltpu.make_async_copy(
            x_ref.at[pl.ds(m - A, A), :], edge_buf.at[1], edge_sems.at[1]
        )
        first_cp.start()
        last_cp.start()

        barrier_sem = pltpu.get_barrier_semaphore()
        for nbr in [left, right]:
            pl.semaphore_signal(
                barrier_sem, inc=1,
                device_id=(nbr,), device_id_type=pl.DeviceIdType.MESH,
            )
        pl.semaphore_wait(barrier_sem, 2)

        first_cp.wait()
        last_cp.wait()
        send_row[0, :, :] = edge_buf[0, pl.ds(0, 1), :]
        send_row[1, :, :] = edge_buf[1, pl.ds(A - 1, 1), :]

        send_r = pltpu.make_async_remote_copy(
            src_ref=send_row.at[1],
            dst_ref=lhalo,
            send_sem=send_sems.at[0],
            recv_sem=recv_sems.at[0],
            device_id=(right,),
            device_id_type=pl.DeviceIdType.MESH,
        )
        send_r.start()
        send_l = pltpu.make_async_remote_copy(
            src_ref=send_row.at[0],
            dst_ref=rhalo,
            send_sem=send_sems.at[1],
            recv_sem=recv_sems.at[1],
            device_id=(left,),
            device_id_type=pl.DeviceIdType.MESH,
        )
        send_l.start()

        def make_in(idx):
            c = order[idx]
            slot = idx % 2
            if c == 0:
                return pltpu.make_async_copy(
                    x_ref.at[pl.ds(0, C + A), :],
                    in_buf.at[slot, pl.ds(0, C + A), :],
                    in_sems.at[slot],
                )
            lo = c * C
            nrows = C + 2 * A if c < N_CHUNKS - 1 else C + A
            return pltpu.make_async_copy(
                x_ref.at[pl.ds(lo - A, nrows), :],
                in_buf.at[slot, pl.ds(0, nrows), :],
                in_sems.at[slot],
            )

        def make_out(idx):
            c = order[idx]
            slot = idx % 2
            return pltpu.make_async_copy(
                stage.at[slot],
                out_ref.at[pl.ds(c * C, C), :],
                out_sems.at[slot],
            )

        make_in(0).start()
        make_in(1).start()

        out_cps = []
        for idx, c in enumerate(order):
            slot = idx % 2
            if idx >= 2:
                out_cps[idx - 2].wait()
            make_in(idx).wait()
            if c == 0:
                send_r.wait_recv()
                row0 = (
                    0.25 * lhalo[:, :]
                    + 0.5 * in_buf[slot, pl.ds(0, 1), :]
                    + 0.25 * in_buf[slot, pl.ds(1, 1), :]
                )
                stage[slot, pl.ds(0, 1), :] = jnp.where(
                    my == 0, in_buf[slot, pl.ds(0, 1), :], row0
                )
                stage[slot, pl.ds(1, C - 1), :] = (
                    0.25 * in_buf[slot, pl.ds(0, C - 1), :]
                    + 0.5 * in_buf[slot, pl.ds(1, C - 1), :]
                    + 0.25 * in_buf[slot, pl.ds(2, C - 1), :]
                )
            elif c == N_CHUNKS - 1:
                send_l.wait_recv()
                stage[slot, pl.ds(0, C - 1), :] = (
                    0.25 * in_buf[slot, pl.ds(A - 1, C - 1), :]
                    + 0.5 * in_buf[slot, pl.ds(A, C - 1), :]
                    + 0.25 * in_buf[slot, pl.ds(A + 1, C - 1), :]
                )
                rowm = (
                    0.25 * in_buf[slot, pl.ds(C + A - 2, 1), :]
                    + 0.5 * in_buf[slot, pl.ds(C + A - 1, 1), :]
                    + 0.25 * rhalo[:, :]
                )
                stage[slot, pl.ds(C - 1, 1), :] = jnp.where(
                    my == N_DEV - 1, in_buf[slot, pl.ds(C + A - 1, 1), :], rowm
                )
            else:
                stage[slot, :, :] = (
                    0.25 * in_buf[slot, pl.ds(A - 1, C), :]
                    + 0.5 * in_buf[slot, pl.ds(A, C), :]
                    + 0.25 * in_buf[slot, pl.ds(A + 1, C), :]
                )
            cp = make_out(idx)
            cp.start()
            out_cps.append(cp)
            if idx + 2 < N_CHUNKS:
                make_in(idx + 2).start()

        out_cps[-2].wait()
        out_cps[-1].wait()
        send_r.wait_send()
        send_l.wait_send()

    return pl.pallas_call(
        body,
        out_shape=jax.ShapeDtypeStruct((m, n), x.dtype),
        in_specs=[pl.BlockSpec(memory_space=pltpu.MemorySpace.HBM)],
        out_specs=pl.BlockSpec(memory_space=pltpu.MemorySpace.HBM),
        scratch_shapes=[
            pltpu.VMEM((1, n), x.dtype),
            pltpu.VMEM((1, n), x.dtype),
            pltpu.VMEM((2, C + 2 * A, n), x.dtype),
            pltpu.VMEM((2, C, n), x.dtype),
            pltpu.VMEM((2, A, n), x.dtype),
            pltpu.VMEM((2, 1, n), x.dtype),
            pltpu.SemaphoreType.DMA((2,)),
            pltpu.SemaphoreType.DMA((2,)),
            pltpu.SemaphoreType.DMA((2,)),
            pltpu.SemaphoreType.DMA((2,)),
            pltpu.SemaphoreType.DMA((2,)),
        ],
        compiler_params=pltpu.CompilerParams(
            collective_id=0,
            vmem_limit_bytes=63 * 1024 * 1024,
        ),
    )(x)


# device time: 47758 ns/iter; 1.0809x vs baseline; 1.0605x over previous
import jax
import jax.numpy as jnp
from jax import lax
from jax.experimental import pallas as pl
from jax.experimental.pallas import tpu as pltpu

N_DEV = 4
N_CHUNKS = 16
NSLOTS = 4
A = 8


def kernel(x):
    m, n = x.shape
    C = m // N_CHUNKS
    order = list(range(1, N_CHUNKS - 1)) + [0, N_CHUNKS - 1]

    def body(x_ref, out_ref, lhalo, rhalo, in_buf, stage, edge_buf, send_row,
             send_sems, recv_sems, in_sems, out_sems, edge_sems):
        my = lax.axis_index("i")
        left = (my - 1) % N_DEV
        right = (my + 1) % N_DEV

        first_cp = pltpu.make_async_copy(
            x_ref.at[pl.ds(0, A), :], edge_buf.at[0], edge_sems.at[0]
        )
        last_cp = pltpu.make_async_copy(
            x_ref.at[pl.ds(m - A, A), :], edge_buf.at[1], edge_sems.at[1]
        )
        first_cp.start()
        last_cp.start()

        barrier_sem = pltpu.get_barrier_semaphore()
        for nbr in [left, right]:
            pl.semaphore_signal(
                barrier_sem, inc=1,
                device_id=(nbr,), device_id_type=pl.DeviceIdType.MESH,
            )
        pl.semaphore_wait(barrier_sem, 2)

        first_cp.wait()
        last_cp.wait()
        send_row[0, :, :] = edge_buf[0, pl.ds(0, 1), :]
        send_row[1, :, :] = edge_buf[1, pl.ds(A - 1, 1), :]

        send_r = pltpu.make_async_remote_copy(
            src_ref=send_row.at[1],
            dst_ref=lhalo,
            send_sem=send_sems.at[0],
            recv_sem=recv_sems.at[0],
            device_id=(right,),
            device_id_type=pl.DeviceIdType.MESH,
        )
        send_r.start()
        send_l = pltpu.make_async_remote_copy(
            src_ref=send_row.at[0],
            dst_ref=rhalo,
            send_sem=send_sems.at[1],
            recv_sem=recv_sems.at[1],
            device_id=(left,),
            device_id_type=pl.DeviceIdType.MESH,
        )
        send_l.start()

        def make_in(idx):
            c = order[idx]
            slot = idx % NSLOTS
            if c == 0:
                return pltpu.make_async_copy(
                    x_ref.at[pl.ds(0, C + A), :],
                    in_buf.at[slot, pl.ds(0, C + A), :],
                    in_sems.at[slot],
                )
            lo = c * C
            nrows = C + 2 * A if c < N_CHUNKS - 1 else C + A
            return pltpu.make_async_copy(
                x_ref.at[pl.ds(lo - A, nrows), :],
                in_buf.at[slot, pl.ds(0, nrows), :],
                in_sems.at[slot],
            )

        def make_out(idx):
            c = order[idx]
            slot = idx % NSLOTS
            return pltpu.make_async_copy(
                stage.at[slot],
                out_ref.at[pl.ds(c * C, C), :],
                out_sems.at[slot],
            )

        for i in range(NSLOTS):
            make_in(i).start()

        out_cps = []
        for idx, c in enumerate(order):
            slot = idx % NSLOTS
            if idx >= NSLOTS:
                out_cps[idx - NSLOTS].wait()
            make_in(idx).wait()
            if c == 0:
                send_r.wait_recv()
                row0 = (
                    0.25 * lhalo[:, :]
                    + 0.5 * in_buf[slot, pl.ds(0, 1), :]
                    + 0.25 * in_buf[slot, pl.ds(1, 1), :]
                )
                stage[slot, pl.ds(0, 1), :] = jnp.where(
                    my == 0, in_buf[slot, pl.ds(0, 1), :], row0
                )
                stage[slot, pl.ds(1, C - 1), :] = (
                    0.25 * in_buf[slot, pl.ds(0, C - 1), :]
                    + 0.5 * in_buf[slot, pl.ds(1, C - 1), :]
                    + 0.25 * in_buf[slot, pl.ds(2, C - 1), :]
                )
            elif c == N_CHUNKS - 1:
                send_l.wait_recv()
                stage[slot, pl.ds(0, C - 1), :] = (
                    0.25 * in_buf[slot, pl.ds(A - 1, C - 1), :]
                    + 0.5 * in_buf[slot, pl.ds(A, C - 1), :]
                    + 0.25 * in_buf[slot, pl.ds(A + 1, C - 1), :]
                )
                rowm = (
                    0.25 * in_buf[slot, pl.ds(C + A - 2, 1), :]
                    + 0.5 * in_buf[slot, pl.ds(C + A - 1, 1), :]
                    + 0.25 * rhalo[:, :]
                )
                stage[slot, pl.ds(C - 1, 1), :] = jnp.where(
                    my == N_DEV - 1, in_buf[slot, pl.ds(C + A - 1, 1), :], rowm
                )
            else:
                stage[slot, :, :] = (
                    0.25 * in_buf[slot, pl.ds(A - 1, C), :]
                    + 0.5 * in_buf[slot, pl.ds(A, C), :]
                    + 0.25 * in_buf[slot, pl.ds(A + 1, C), :]
                )
            cp = make_out(idx)
            cp.start()
            out_cps.append(cp)
            if idx + NSLOTS < N_CHUNKS:
                make_in(idx + NSLOTS).start()

        for cp in out_cps[-NSLOTS:]:
            cp.wait()
        send_r.wait_send()
        send_l.wait_send()

    return pl.pallas_call(
        body,
        out_shape=jax.ShapeDtypeStruct((m, n), x.dtype),
        in_specs=[pl.BlockSpec(memory_space=pltpu.MemorySpace.HBM)],
        out_specs=pl.BlockSpec(memory_space=pltpu.MemorySpace.HBM),
        scratch_shapes=[
            pltpu.VMEM((1, n), x.dtype),
            pltpu.VMEM((1, n), x.dtype),
            pltpu.VMEM((NSLOTS, C + 2 * A, n), x.dtype),
            pltpu.VMEM((NSLOTS, C, n), x.dtype),
            pltpu.VMEM((2, A, n), x.dtype),
            pltpu.VMEM((2, 1, n), x.dtype),
            pltpu.SemaphoreType.DMA((2,)),
            pltpu.SemaphoreType.DMA((2,)),
            pltpu.SemaphoreType.DMA((NSLOTS,)),
            pltpu.SemaphoreType.DMA((NSLOTS,)),
            pltpu.SemaphoreType.DMA((2,)),
        ],
        compiler_params=pltpu.CompilerParams(
            collective_id=0,
            vmem_limit_bytes=63 * 1024 * 1024,
        ),
    )(x)
